# baseline (speedup 1.0000x reference)
import functools

import jax
import jax.numpy as jnp
from jax import lax
from jax.experimental import pallas as pl
from jax.experimental.pallas import tpu as pltpu
from jax.experimental.pallas import tpu_sc as plsc

N_NODES = 100000
N_FEAT = 128
N_GRAPHS = 64
NW = 32
BS = 400
NBLK = N_NODES // BS
NLANE = 16
NJ = N_FEAT // NLANE


def _sc_body(x_hbm, batch_hbm, out_hbm, xbuf, ibuf, acc):
    cid = lax.axis_index("c")
    sid = lax.axis_index("s")
    wid = sid * 2 + cid

    zero = jnp.zeros((NLANE,), jnp.float32)

    def zero_body(i, _):
        for j in range(NJ):
            acc[i, pl.ds(j * NLANE, NLANE)] = zero
        return 0

    lax.fori_loop(0, N_GRAPHS, zero_body, 0)

    def blk_body(t, carry):
        b = wid + t * NW
        pltpu.sync_copy(x_hbm.at[pl.ds(b * BS, BS)], xbuf)
        pltpu.sync_copy(batch_hbm.at[pl.ds(b * BS, BS)], ibuf)

        def grp_body(g, gcarry):
            s_cur = gcarry[0]
            vs = list(gcarry[1:])
            segv = ibuf[pl.ds(g * NLANE, NLANE)]
            for l in range(NLANE):
                s = segv[l]
                changed = s != s_cur
                vs_now = tuple(vs)
                s_now = s_cur

                @pl.when(changed)
                def _flush(vs_now=vs_now, s_now=s_now):
                    for j in range(NJ):
                        sl = pl.ds(j * NLANE, NLANE)
                        acc[s_now, sl] = acc[s_now, sl] + vs_now[j]

                r = g * NLANE + l
                for j in range(NJ):
                    vs[j] = jnp.where(changed, zero, vs[j]) + xbuf[
                        r, pl.ds(j * NLANE, NLANE)
                    ]
                s_cur = s
            return (s_cur,) + tuple(vs)

        return lax.fori_loop(0, BS // NLANE, grp_body, carry)

    trips = jnp.where(wid <= (NBLK % NW) - 1, NBLK // NW + 1, NBLK // NW)
    init = (jnp.int32(0),) + tuple(zero for _ in range(NJ))
    carry = lax.fori_loop(0, trips, blk_body, init)

    s_cur = carry[0]
    for j in range(NJ):
        sl = pl.ds(j * NLANE, NLANE)
        acc[s_cur, sl] = acc[s_cur, sl] + carry[1 + j]

    pltpu.sync_copy(acc, out_hbm.at[wid])


def _sc_segment_sum(x, batch):
    mesh = plsc.VectorSubcoreMesh(core_axis_name="c", subcore_axis_name="s")
    f = functools.partial(
        pl.kernel,
        mesh=mesh,
        out_type=jax.ShapeDtypeStruct((NW, N_GRAPHS, N_FEAT), jnp.float32),
        scratch_types=[
            pltpu.VMEM((BS, N_FEAT), jnp.float32),
            pltpu.VMEM((BS,), jnp.int32),
            pltpu.VMEM((N_GRAPHS, N_FEAT), jnp.float32),
        ],
    )(_sc_body)
    return f(x, batch)


def _tc_tail_body(p_ref, filt_ref, w_ref, b_ref, out_ref):
    g = jnp.sum(p_ref[...], axis=0)
    sx = g * filt_ref[...]
    y = lax.dot_general(
        sx, w_ref[...], (((1,), (1,)), ((), ())),
        preferred_element_type=jnp.float32,
        precision=lax.Precision.HIGHEST,
    )
    out_ref[...] = jnp.tanh(y + b_ref[...])


def _tc_tail(partials, filt, W, b):
    return pl.pallas_call(
        _tc_tail_body,
        out_shape=jax.ShapeDtypeStruct((N_GRAPHS, N_FEAT), jnp.float32),
    )(partials, filt.reshape(1, N_FEAT), W, b.reshape(1, N_FEAT))


def kernel(x, batch, filt, W, b):
    partials = _sc_segment_sum(x, batch.astype(jnp.int32))
    return _tc_tail(partials, filt, W, b)


# speedup vs baseline: 1.1981x; 1.1981x over previous
import functools

import jax
import jax.numpy as jnp
from jax import lax
from jax.experimental import pallas as pl
from jax.experimental.pallas import tpu as pltpu
from jax.experimental.pallas import tpu_sc as plsc

N_NODES = 100000
N_FEAT = 128
N_GRAPHS = 64
NW = 32
BS = 400
NBLK = N_NODES // BS
NLANE = 16
NJ = N_FEAT // NLANE
NGRP = BS // NLANE


def _tree_sum(vals):
    vals = list(vals)
    while len(vals) > 1:
        nxt = [vals[i] + vals[i + 1] for i in range(0, len(vals) - 1, 2)]
        if len(vals) % 2:
            nxt.append(vals[-1])
        vals = nxt
    return vals[0]


def _sc_body(
    x_hbm, batch_hbm, out_hbm,
    xbuf0, xbuf1, ibuf0, ibuf1, acc, cur,
    semx0, semx1, semi0, semi1,
):
    cid = lax.axis_index("c")
    sid = lax.axis_index("s")
    wid = sid * 2 + cid

    zero = jnp.zeros((NLANE,), jnp.float32)

    def zero_body(i, _):
        for j in range(NJ):
            acc[i, pl.ds(j * NLANE, NLANE)] = zero
        return 0

    lax.fori_loop(0, N_GRAPHS, zero_body, 0)
    for j in range(NJ):
        cur[pl.ds(j * NLANE, NLANE)] = zero

    trips = jnp.where(wid <= (NBLK % NW) - 1, NBLK // NW + 1, NBLK // NW)

    def start_dma(t, xb, ib, sx, si):
        b = wid + t * NW
        pltpu.async_copy(x_hbm.at[pl.ds(b * BS, BS)], xb, sx)
        pltpu.async_copy(batch_hbm.at[pl.ds(b * BS, BS)], ib, si)

    def wait_dma(xb, ib, sx, si):
        pltpu.make_async_copy(x_hbm.at[pl.ds(0, BS)], xb, sx).wait()
        pltpu.make_async_copy(batch_hbm.at[pl.ds(0, BS)], ib, si).wait()

    def process(xb, ib, s_init):
        def grp_body(g, s_cur):
            segv = ib[pl.ds(g * NLANE, NLANE)]
            s_last = segv[NLANE - 1]
            uniform = (segv[0] == s_cur) & (s_last == s_cur)

            @pl.when(uniform)
            def _fast():
                for j in range(NJ):
                    sl = pl.ds(j * NLANE, NLANE)
                    gsum = _tree_sum(
                        [xb[g * NLANE + l, sl] for l in range(NLANE)]
                    )
                    cur[sl] = cur[sl] + gsum

            @pl.when(jnp.logical_not(uniform))
            def _slow():
                s_c = s_cur
                for l in range(NLANE):
                    s = segv[l]
                    changed = s != s_c

                    @pl.when(changed)
                    def _flush(s_now=s_c):
                        for j in range(NJ):
                            sl = pl.ds(j * NLANE, NLANE)
                            acc[s_now, sl] = acc[s_now, sl] + cur[sl]
                            cur[sl] = zero

                    r = g * NLANE + l
                    for j in range(NJ):
                        sl = pl.ds(j * NLANE, NLANE)
                        cur[sl] = cur[sl] + xb[r, sl]
                    s_c = s

            return s_last

        lax.fori_loop(0, NGRP, grp_body, s_init)

    start_dma(0, xbuf0, ibuf0, semx0, semi0)

    def blk_body(t, s_cur):
        p = lax.rem(t, 2)

        @pl.when((t + 1 < trips) & (p == 0))
        def _pf0():
            start_dma(t + 1, xbuf1, ibuf1, semx1, semi1)

        @pl.when((t + 1 < trips) & (p == 1))
        def _pf1():
            start_dma(t + 1, xbuf0, ibuf0, semx0, semi0)

        @pl.when(p == 0)
        def _do0():
            wait_dma(xbuf0, ibuf0, semx0, semi0)
            process(xbuf0, ibuf0, s_cur)

        @pl.when(p == 1)
        def _do1():
            wait_dma(xbuf1, ibuf1, semx1, semi1)
            process(xbuf1, ibuf1, s_cur)

        last0 = ibuf0[pl.ds(BS - NLANE, NLANE)][NLANE - 1]
        last1 = ibuf1[pl.ds(BS - NLANE, NLANE)][NLANE - 1]
        return jnp.where(p == 0, last0, last1)

    s_fin = lax.fori_loop(0, trips, blk_body, jnp.int32(0))

    for j in range(NJ):
        sl = pl.ds(j * NLANE, NLANE)
        acc[s_fin, sl] = acc[s_fin, sl] + cur[sl]

    pltpu.sync_copy(acc, out_hbm.at[wid])


def _sc_segment_sum(x, batch):
    mesh = plsc.VectorSubcoreMesh(core_axis_name="c", subcore_axis_name="s")
    f = functools.partial(
        pl.kernel,
        mesh=mesh,
        out_type=jax.ShapeDtypeStruct((NW, N_GRAPHS, N_FEAT), jnp.float32),
        scratch_types=[
            pltpu.VMEM((BS, N_FEAT), jnp.float32),
            pltpu.VMEM((BS, N_FEAT), jnp.float32),
            pltpu.VMEM((BS,), jnp.int32),
            pltpu.VMEM((BS,), jnp.int32),
            pltpu.VMEM((N_GRAPHS, N_FEAT), jnp.float32),
            pltpu.VMEM((N_FEAT,), jnp.float32),
            pltpu.SemaphoreType.DMA,
            pltpu.SemaphoreType.DMA,
            pltpu.SemaphoreType.DMA,
            pltpu.SemaphoreType.DMA,
        ],
    )(_sc_body)
    return f(x, batch)


def _tc_tail_body(p_ref, filt_ref, w_ref, b_ref, out_ref):
    g = jnp.sum(p_ref[...], axis=0)
    sx = g * filt_ref[...]
    y = lax.dot_general(
        sx, w_ref[...], (((1,), (1,)), ((), ())),
        preferred_element_type=jnp.float32,
        precision=lax.Precision.HIGHEST,
    )
    out_ref[...] = jnp.tanh(y + b_ref[...])


def _tc_tail(partials, filt, W, b):
    return pl.pallas_call(
        _tc_tail_body,
        out_shape=jax.ShapeDtypeStruct((N_GRAPHS, N_FEAT), jnp.float32),
    )(partials, filt.reshape(1, N_FEAT), W, b.reshape(1, N_FEAT))


def kernel(x, batch, filt, W, b):
    partials = _sc_segment_sum(x, batch.astype(jnp.int32))
    return _tc_tail(partials, filt, W, b)


# speedup vs baseline: 1.2238x; 1.0214x over previous
import functools

import jax
import jax.numpy as jnp
from jax import lax
from jax.experimental import pallas as pl
from jax.experimental.pallas import tpu as pltpu
from jax.experimental.pallas import tpu_sc as plsc

N_NODES = 100000
N_FEAT = 128
N_GRAPHS = 64
NW = 32
BS = 160
NBLK = N_NODES // BS
NLANE = 16
NJ = N_FEAT // NLANE
NGRP = BS // NLANE
NREM = BS - NGRP * NLANE
NBUF = 4


def _tree_sum(vals):
    vals = list(vals)
    while len(vals) > 1:
        nxt = [vals[i] + vals[i + 1] for i in range(0, len(vals) - 1, 2)]
        if len(vals) % 2:
            nxt.append(vals[-1])
        vals = nxt
    return vals[0]


def _sc_body(x_hbm, batch_hbm, out_hbm, *refs):
    xbufs = refs[0:NBUF]
    ibufs = refs[NBUF : 2 * NBUF]
    acc = refs[2 * NBUF]
    semx = refs[2 * NBUF + 1 : 2 * NBUF + 1 + NBUF]
    semi = refs[2 * NBUF + 1 + NBUF : 2 * NBUF + 1 + 2 * NBUF]

    cid = lax.axis_index("c")
    sid = lax.axis_index("s")
    wid = sid * 2 + cid

    zero = jnp.zeros((NLANE,), jnp.float32)

    def zero_body(i, _):
        for j in range(NJ):
            acc[i, pl.ds(j * NLANE, NLANE)] = zero
        return 0

    lax.fori_loop(0, N_GRAPHS, zero_body, 0)

    trips = jnp.where(wid <= (NBLK % NW) - 1, NBLK // NW + 1, NBLK // NW)

    def start_dma(t, s):
        b = wid + t * NW
        pltpu.async_copy(x_hbm.at[pl.ds(b * BS, BS)], xbufs[s], semx[s])
        pltpu.async_copy(batch_hbm.at[pl.ds(b * BS, BS)], ibufs[s], semi[s])

    def wait_dma(s):
        pltpu.make_async_copy(
            x_hbm.at[pl.ds(0, BS)], xbufs[s], semx[s]
        ).wait()
        pltpu.make_async_copy(
            batch_hbm.at[pl.ds(0, BS)], ibufs[s], semi[s]
        ).wait()

    def process(xb, ib, s_init):
        def grp_body(g, s_cur):
            segv = ib[pl.ds(g * NLANE, NLANE)]
            s_last = segv[NLANE - 1]
            uniform = (segv[0] == s_cur) & (s_last == s_cur)

            @pl.when(uniform)
            def _fast():
                for j in range(NJ):
                    sl = pl.ds(j * NLANE, NLANE)
                    gsum = _tree_sum(
                        [xb[g * NLANE + l, sl] for l in range(NLANE)]
                    )
                    acc[s_cur, sl] = acc[s_cur, sl] + gsum

            @pl.when(jnp.logical_not(uniform))
            def _slow():
                for l in range(NLANE):
                    s = segv[l]
                    r = g * NLANE + l
                    for j in range(NJ):
                        sl = pl.ds(j * NLANE, NLANE)
                        acc[s, sl] = acc[s, sl] + xb[r, sl]

            return s_last

        s_c = lax.fori_loop(0, NGRP, grp_body, s_init)
        if NREM:
            segv = ib[pl.ds(NGRP * NLANE, NLANE)]
            for l in range(NREM):
                s = segv[l]
                r = NGRP * NLANE + l
                for j in range(NJ):
                    sl = pl.ds(j * NLANE, NLANE)
                    acc[s, sl] = acc[s, sl] + xb[r, sl]
        return s_c

    for s in range(NBUF - 1):
        start_dma(s, s)

    def blk_body(t, s_cur):
        p = lax.rem(t, NBUF)

        for s in range(NBUF):

            @pl.when((t + NBUF - 1 < trips) & (p == s))
            def _pf(s=s):
                start_dma(t + NBUF - 1, (s + NBUF - 1) % NBUF)

        for s in range(NBUF):

            @pl.when(p == s)
            def _do(s=s):
                wait_dma(s)
                process(xbufs[s], ibufs[s], s_cur)

        lasts = [
            ibufs[s][pl.ds(BS - NLANE, NLANE)][NLANE - 1] for s in range(NBUF)
        ]
        s_new = lasts[NBUF - 1]
        for s in range(NBUF - 2, -1, -1):
            s_new = jnp.where(p == s, lasts[s], s_new)
        return s_new

    lax.fori_loop(0, trips, blk_body, jnp.int32(0))

    pltpu.sync_copy(acc, out_hbm.at[wid])


def _sc_segment_sum(x, batch):
    mesh = plsc.VectorSubcoreMesh(core_axis_name="c", subcore_axis_name="s")
    scratch = (
        [pltpu.VMEM((BS, N_FEAT), jnp.float32) for _ in range(NBUF)]
        + [pltpu.VMEM((BS,), jnp.int32) for _ in range(NBUF)]
        + [pltpu.VMEM((N_GRAPHS, N_FEAT), jnp.float32)]
        + [pltpu.SemaphoreType.DMA for _ in range(2 * NBUF)]
    )
    f = functools.partial(
        pl.kernel,
        mesh=mesh,
        out_type=jax.ShapeDtypeStruct((NW, N_GRAPHS, N_FEAT), jnp.float32),
        scratch_types=scratch,
    )(_sc_body)
    return f(x, batch)


def _tc_tail_body(p_ref, filt_ref, w_ref, b_ref, out_ref):
    g = jnp.sum(p_ref[...], axis=0)
    sx = g * filt_ref[...]
    y = lax.dot_general(
        sx, w_ref[...], (((1,), (1,)), ((), ())),
        preferred_element_type=jnp.float32,
        precision=lax.Precision.HIGHEST,
    )
    out_ref[...] = jnp.tanh(y + b_ref[...])


def _tc_tail(partials, filt, W, b):
    return pl.pallas_call(
        _tc_tail_body,
        out_shape=jax.ShapeDtypeStruct((N_GRAPHS, N_FEAT), jnp.float32),
    )(partials, filt.reshape(1, N_FEAT), W, b.reshape(1, N_FEAT))


def kernel(x, batch, filt, W, b):
    partials = _sc_segment_sum(x, batch.astype(jnp.int32))
    return _tc_tail(partials, filt, W, b)


# speedup vs baseline: 1.6988x; 1.3881x over previous
import functools

import jax
import jax.numpy as jnp
from jax import lax
from jax.experimental import pallas as pl
from jax.experimental.pallas import tpu as pltpu
from jax.experimental.pallas import tpu_sc as plsc

N_NODES = 100000
N_FEAT = 128
N_GRAPHS = 64
NW = 32

TC_BS = 2000
R_TC = 52000
NB_TC = R_TC // TC_BS

BS = 160
SC_BASE_BLK = R_TC // BS
NBLK = (N_NODES - R_TC) // BS
NLANE = 16
NJ = N_FEAT // NLANE
NGRP = BS // NLANE
NBUF = 4


def _tree_sum(vals):
    vals = list(vals)
    while len(vals) > 1:
        nxt = [vals[i] + vals[i + 1] for i in range(0, len(vals) - 1, 2)]
        if len(vals) % 2:
            nxt.append(vals[-1])
        vals = nxt
    return vals[0]


def _sc_body(x_hbm, batch_hbm, out_hbm, *refs):
    xbufs = refs[0:NBUF]
    ibufs = refs[NBUF : 2 * NBUF]
    acc = refs[2 * NBUF]
    semx = refs[2 * NBUF + 1 : 2 * NBUF + 1 + NBUF]
    semi = refs[2 * NBUF + 1 + NBUF : 2 * NBUF + 1 + 2 * NBUF]

    cid = lax.axis_index("c")
    sid = lax.axis_index("s")
    wid = sid * 2 + cid

    zero = jnp.zeros((NLANE,), jnp.float32)

    def zero_body(i, _):
        for j in range(NJ):
            acc[i, pl.ds(j * NLANE, NLANE)] = zero
        return 0

    lax.fori_loop(0, N_GRAPHS, zero_body, 0)

    trips = jnp.where(wid <= (NBLK % NW) - 1, NBLK // NW + 1, NBLK // NW)

    def start_dma(t, s):
        b = SC_BASE_BLK + wid + t * NW
        pltpu.async_copy(x_hbm.at[pl.ds(b * BS, BS)], xbufs[s], semx[s])
        pltpu.async_copy(batch_hbm.at[pl.ds(b * BS, BS)], ibufs[s], semi[s])

    def wait_dma(s):
        pltpu.make_async_copy(
            x_hbm.at[pl.ds(0, BS)], xbufs[s], semx[s]
        ).wait()
        pltpu.make_async_copy(
            batch_hbm.at[pl.ds(0, BS)], ibufs[s], semi[s]
        ).wait()

    def process(xb, ib, s_init):
        def grp_body(g, s_cur):
            segv = ib[pl.ds(g * NLANE, NLANE)]
            s_last = segv[NLANE - 1]
            uniform = (segv[0] == s_cur) & (s_last == s_cur)

            @pl.when(uniform)
            def _fast():
                for j in range(NJ):
                    sl = pl.ds(j * NLANE, NLANE)
                    gsum = _tree_sum(
                        [xb[g * NLANE + l, sl] for l in range(NLANE)]
                    )
                    acc[s_cur, sl] = acc[s_cur, sl] + gsum

            @pl.when(jnp.logical_not(uniform))
            def _slow():
                for l in range(NLANE):
                    s = segv[l]
                    r = g * NLANE + l
                    for j in range(NJ):
                        sl = pl.ds(j * NLANE, NLANE)
                        acc[s, sl] = acc[s, sl] + xb[r, sl]

            return s_last

        return lax.fori_loop(0, NGRP, grp_body, s_init)

    for s in range(NBUF - 1):
        start_dma(s, s)

    def blk_body(t, s_cur):
        p = lax.rem(t, NBUF)

        for s in range(NBUF):

            @pl.when((t + NBUF - 1 < trips) & (p == s))
            def _pf(s=s):
                start_dma(t + NBUF - 1, (s + NBUF - 1) % NBUF)

        for s in range(NBUF):

            @pl.when(p == s)
            def _do(s=s):
                wait_dma(s)
                process(xbufs[s], ibufs[s], s_cur)

        lasts = [
            ibufs[s][pl.ds(BS - NLANE, NLANE)][NLANE - 1] for s in range(NBUF)
        ]
        s_new = lasts[NBUF - 1]
        for s in range(NBUF - 2, -1, -1):
            s_new = jnp.where(p == s, lasts[s], s_new)
        return s_new

    lax.fori_loop(0, trips, blk_body, jnp.int32(0))

    pltpu.sync_copy(acc, out_hbm.at[wid])


def _sc_segment_sum(x, batch):
    mesh = plsc.VectorSubcoreMesh(core_axis_name="c", subcore_axis_name="s")
    scratch = (
        [pltpu.VMEM((BS, N_FEAT), jnp.float32) for _ in range(NBUF)]
        + [pltpu.VMEM((BS,), jnp.int32) for _ in range(NBUF)]
        + [pltpu.VMEM((N_GRAPHS, N_FEAT), jnp.float32)]
        + [pltpu.SemaphoreType.DMA for _ in range(2 * NBUF)]
    )
    f = functools.partial(
        pl.kernel,
        mesh=mesh,
        out_type=jax.ShapeDtypeStruct((NW, N_GRAPHS, N_FEAT), jnp.float32),
        scratch_types=scratch,
    )(_sc_body)
    return f(x, batch)


def _tc_onehot_body(batch_ref, x_ref, out_ref, acc_ref):
    i = pl.program_id(0)
    seg = batch_ref[0]
    oh = (lax.broadcasted_iota(jnp.int32, (N_GRAPHS, TC_BS), 0) == seg).astype(
        jnp.float32
    )
    part = jnp.dot(
        oh, x_ref[...],
        preferred_element_type=jnp.float32,
        precision=lax.Precision.HIGHEST,
    )

    @pl.when(i == 0)
    def _init():
        acc_ref[...] = part

    @pl.when(i != 0)
    def _acc():
        acc_ref[...] += part

    @pl.when(i == NB_TC - 1)
    def _final():
        out_ref[...] = acc_ref[...]


def _tc_onehot(x, batch3):
    return pl.pallas_call(
        _tc_onehot_body,
        grid=(NB_TC,),
        in_specs=[
            pl.BlockSpec((1, 1, TC_BS), lambda i: (i, 0, 0)),
            pl.BlockSpec((TC_BS, N_FEAT), lambda i: (i, 0)),
        ],
        out_specs=pl.BlockSpec((N_GRAPHS, N_FEAT), lambda i: (0, 0)),
        out_shape=jax.ShapeDtypeStruct((N_GRAPHS, N_FEAT), jnp.float32),
        scratch_shapes=[pltpu.VMEM((N_GRAPHS, N_FEAT), jnp.float32)],
    )(batch3, x)


def _tc_tail_body(p_ref, gtc_ref, filt_ref, w_ref, b_ref, out_ref):
    g = jnp.sum(p_ref[...], axis=0) + gtc_ref[...]
    sx = g * filt_ref[...]
    y = lax.dot_general(
        sx, w_ref[...], (((1,), (1,)), ((), ())),
        preferred_element_type=jnp.float32,
        precision=lax.Precision.HIGHEST,
    )
    out_ref[...] = jnp.tanh(y + b_ref[...])


def _tc_tail(partials, g_tc, filt, W, b):
    return pl.pallas_call(
        _tc_tail_body,
        out_shape=jax.ShapeDtypeStruct((N_GRAPHS, N_FEAT), jnp.float32),
    )(partials, g_tc, filt.reshape(1, N_FEAT), W, b.reshape(1, N_FEAT))


def kernel(x, batch, filt, W, b):
    batch = batch.astype(jnp.int32)
    partials = _sc_segment_sum(x, batch)
    batch3 = batch.reshape(N_NODES // TC_BS, 1, TC_BS)
    g_tc = _tc_onehot(x, batch3)
    return _tc_tail(partials, g_tc, filt, W, b)


# speedup vs baseline: 1.7839x; 1.0501x over previous
import functools

import jax
import jax.numpy as jnp
from jax import lax
from jax.experimental import pallas as pl
from jax.experimental.pallas import tpu as pltpu
from jax.experimental.pallas import tpu_sc as plsc

N_NODES = 100000
N_FEAT = 128
N_GRAPHS = 64
NW = 32

TC_BS = 2000
R_TC = 56000
NB_TC = R_TC // TC_BS

BS = 160
SC_BASE_BLK = R_TC // BS
NBLK = (N_NODES - R_TC) // BS
NLANE = 16
NJ = N_FEAT // NLANE
NGRP = BS // NLANE
NBUF = 4


def _tree_sum(vals):
    vals = list(vals)
    while len(vals) > 1:
        nxt = [vals[i] + vals[i + 1] for i in range(0, len(vals) - 1, 2)]
        if len(vals) % 2:
            nxt.append(vals[-1])
        vals = nxt
    return vals[0]


def _sc_body(x_hbm, batch_hbm, out_hbm, *refs):
    xbufs = refs[0:NBUF]
    ibufs = refs[NBUF : 2 * NBUF]
    acc = refs[2 * NBUF]
    semx = refs[2 * NBUF + 1 : 2 * NBUF + 1 + NBUF]
    semi = refs[2 * NBUF + 1 + NBUF : 2 * NBUF + 1 + 2 * NBUF]

    cid = lax.axis_index("c")
    sid = lax.axis_index("s")
    wid = sid * 2 + cid

    zero = jnp.zeros((NLANE,), jnp.float32)

    def zero_body(i, _):
        for j in range(NJ):
            acc[i, pl.ds(j * NLANE, NLANE)] = zero
        return 0

    lax.fori_loop(0, N_GRAPHS, zero_body, 0)

    trips = jnp.where(wid <= (NBLK % NW) - 1, NBLK // NW + 1, NBLK // NW)

    def start_dma(t, s):
        b = SC_BASE_BLK + wid + t * NW
        pltpu.async_copy(x_hbm.at[pl.ds(b * BS, BS)], xbufs[s], semx[s])
        pltpu.async_copy(batch_hbm.at[pl.ds(b * BS, BS)], ibufs[s], semi[s])

    def wait_dma(s):
        pltpu.make_async_copy(
            x_hbm.at[pl.ds(0, BS)], xbufs[s], semx[s]
        ).wait()
        pltpu.make_async_copy(
            batch_hbm.at[pl.ds(0, BS)], ibufs[s], semi[s]
        ).wait()

    def process(xb, ib, s_init):
        def grp_body(g, s_cur):
            segv = ib[pl.ds(g * NLANE, NLANE)]
            s_last = segv[NLANE - 1]
            uniform = (segv[0] == s_cur) & (s_last == s_cur)

            @pl.when(uniform)
            def _fast():
                for j in range(NJ):
                    sl = pl.ds(j * NLANE, NLANE)
                    gsum = _tree_sum(
                        [xb[g * NLANE + l, sl] for l in range(NLANE)]
                    )
                    acc[s_cur, sl] = acc[s_cur, sl] + gsum

            @pl.when(jnp.logical_not(uniform))
            def _slow():
                for l in range(NLANE):
                    s = segv[l]
                    r = g * NLANE + l
                    for j in range(NJ):
                        sl = pl.ds(j * NLANE, NLANE)
                        acc[s, sl] = acc[s, sl] + xb[r, sl]

            return s_last

        return lax.fori_loop(0, NGRP, grp_body, s_init)

    for s in range(NBUF - 1):
        start_dma(s, s)

    def blk_body(t, s_cur):
        p = lax.rem(t, NBUF)

        for s in range(NBUF):

            @pl.when((t + NBUF - 1 < trips) & (p == s))
            def _pf(s=s):
                start_dma(t + NBUF - 1, (s + NBUF - 1) % NBUF)

        for s in range(NBUF):

            @pl.when(p == s)
            def _do(s=s):
                wait_dma(s)
                process(xbufs[s], ibufs[s], s_cur)

        lasts = [
            ibufs[s][pl.ds(BS - NLANE, NLANE)][NLANE - 1] for s in range(NBUF)
        ]
        s_new = lasts[NBUF - 1]
        for s in range(NBUF - 2, -1, -1):
            s_new = jnp.where(p == s, lasts[s], s_new)
        return s_new

    lax.fori_loop(0, trips, blk_body, jnp.int32(0))

    pltpu.sync_copy(acc, out_hbm.at[wid])


def _sc_segment_sum(x, batch):
    mesh = plsc.VectorSubcoreMesh(core_axis_name="c", subcore_axis_name="s")
    scratch = (
        [pltpu.VMEM((BS, N_FEAT), jnp.float32) for _ in range(NBUF)]
        + [pltpu.VMEM((BS,), jnp.int32) for _ in range(NBUF)]
        + [pltpu.VMEM((N_GRAPHS, N_FEAT), jnp.float32)]
        + [pltpu.SemaphoreType.DMA for _ in range(2 * NBUF)]
    )
    f = functools.partial(
        pl.kernel,
        mesh=mesh,
        out_type=jax.ShapeDtypeStruct((NW, N_GRAPHS, N_FEAT), jnp.float32),
        scratch_types=scratch,
    )(_sc_body)
    return f(x, batch)


def _tc_onehot_body(batch_ref, x_ref, out_ref, acc_ref):
    i = pl.program_id(0)
    seg = batch_ref[0]
    oh = (lax.broadcasted_iota(jnp.int32, (N_GRAPHS, TC_BS), 0) == seg).astype(
        jnp.float32
    )
    part = jnp.dot(
        oh, x_ref[...],
        preferred_element_type=jnp.float32,
        precision=lax.Precision.HIGHEST,
    )

    @pl.when(i == 0)
    def _init():
        acc_ref[...] = part

    @pl.when(i != 0)
    def _acc():
        acc_ref[...] += part

    @pl.when(i == NB_TC - 1)
    def _final():
        out_ref[...] = acc_ref[...]


def _tc_onehot(x, batch3):
    return pl.pallas_call(
        _tc_onehot_body,
        grid=(NB_TC,),
        in_specs=[
            pl.BlockSpec((1, 1, TC_BS), lambda i: (i, 0, 0)),
            pl.BlockSpec((TC_BS, N_FEAT), lambda i: (i, 0)),
        ],
        out_specs=pl.BlockSpec((N_GRAPHS, N_FEAT), lambda i: (0, 0)),
        out_shape=jax.ShapeDtypeStruct((N_GRAPHS, N_FEAT), jnp.float32),
        scratch_shapes=[pltpu.VMEM((N_GRAPHS, N_FEAT), jnp.float32)],
    )(batch3, x)


def _tc_tail_body(p_ref, gtc_ref, filt_ref, w_ref, b_ref, out_ref):
    g = jnp.sum(p_ref[...], axis=0) + gtc_ref[...]
    sx = g * filt_ref[...]
    y = lax.dot_general(
        sx, w_ref[...], (((1,), (1,)), ((), ())),
        preferred_element_type=jnp.float32,
        precision=lax.Precision.HIGHEST,
    )
    out_ref[...] = jnp.tanh(y + b_ref[...])


def _tc_tail(partials, g_tc, filt, W, b):
    return pl.pallas_call(
        _tc_tail_body,
        out_shape=jax.ShapeDtypeStruct((N_GRAPHS, N_FEAT), jnp.float32),
    )(partials, g_tc, filt.reshape(1, N_FEAT), W, b.reshape(1, N_FEAT))


def kernel(x, batch, filt, W, b):
    batch = batch.astype(jnp.int32)
    partials = _sc_segment_sum(x, batch)
    batch3 = batch.reshape(N_NODES // TC_BS, 1, TC_BS)
    g_tc = _tc_onehot(x, batch3)
    return _tc_tail(partials, g_tc, filt, W, b)
